# Initial kernel scaffold; baseline (speedup 1.0000x reference)
import jax
import jax.numpy as jnp
from jax.experimental import pallas as pl


def kernel(x, mask, pe_k, pe_v, bases, comp, root, bias, W1, W2, Wp):
    raise NotImplementedError("write your pallas kernel here")


# speedup vs baseline: 6.4740x; 6.4740x over previous
import functools

import jax
import jax.numpy as jnp
import numpy as np
from jax.experimental import pallas as pl
from jax.experimental.pallas import tpu as pltpu

_WINDOW = 7
_REL_NUM = _WINDOW + 2
_MAX_LEN = 10
_PPAD = 16

_hdot = functools.partial(jnp.dot, precision=jax.lax.Precision.HIGHEST)


@functools.lru_cache(maxsize=None)
def _static_graph(L):
    i = np.arange(L)[:, None]
    j = np.arange(L)[None, :]
    rel_adj = np.where(j > i, 1, 0).astype(np.int64)
    d = i - j
    lower = -np.minimum(np.ceil(d / 2.0), float(_WINDOW + 1)).astype(np.int64)
    rel_adj = np.where(j < i, lower, rel_adj)
    et = (rel_adj % _REL_NUM).astype(np.int64)
    et_in = et.T
    G = np.zeros((_REL_NUM, L, L), np.float32)
    for r in range(_REL_NUM):
        sel = (et_in == r)
        cnt = np.maximum(sel.sum(axis=1, keepdims=True), 1)
        G[r] = sel / cnt
    pm = np.clip(i - j + 1, 0, _MAX_LEN).reshape(-1)
    pmoh = np.zeros((L * L, _PPAD), np.float32)
    pmoh[np.arange(L * L), pm] = 1.0
    return G, pmoh


def _body(TI, L, D, H,
          x_ref, mask_ref, comp_ref, g_ref, b0_ref, b1_ref, root_ref,
          bias_ref, pek_ref, pev_ref, w1a_ref, w1b_ref, w1c_ref, w1d_ref,
          w2_ref, wp_ref, pmoh_ref, o_ref, s_s, t_s, rtab_s):
    t = pl.program_id(1)

    @pl.when(t == 0)
    def _per_batch():
        xb = x_ref[0]
        xb0 = _hdot(xb, b0_ref[:])
        xb1 = _hdot(xb, b1_ref[:])
        m0 = comp_ref[0:1, 0:1] * g_ref[0]
        m1 = comp_ref[0:1, 1:2] * g_ref[0]
        for r in range(1, _REL_NUM):
            m0 = m0 + comp_ref[r:r + 1, 0:1] * g_ref[r]
            m1 = m1 + comp_ref[r:r + 1, 1:2] * g_ref[r]
        outb = (_hdot(m0, xb0) + _hdot(m1, xb1)
                + _hdot(xb, root_ref[:]) + bias_ref[:])
        s_s[:] = _hdot(outb, w1a_ref[:])
        t_s[:] = _hdot(outb, w1c_ref[:])
        rtab_s[:] = _hdot(pek_ref[:], w1b_ref[:]) + _hdot(pev_ref[:], w1d_ref[:])

    rexp = _hdot(pmoh_ref[:], rtab_s[:])
    t_tile = t_s[pl.ds(t * TI, TI), :]
    pre = (jnp.broadcast_to(s_s[:][None, :, :], (TI, L, H))
           + jnp.broadcast_to(t_tile[:, None, :], (TI, L, H)))
    pre = pre.reshape(TI * L, H) + rexp
    h1 = jnp.maximum(pre, 0.0)
    h2 = jnp.maximum(_hdot(h1, w2_ref[:]), 0.0)
    s = _hdot(h2, wp_ref[:])
    o_ref[0] = jax.nn.sigmoid(s) * mask_ref[0]


def kernel(x, mask, pe_k, pe_v, bases, comp, root, bias, W1, W2, Wp):
    B, L, D = x.shape
    H = W2.shape[0]
    P = pe_k.shape[1]
    TI = 32
    NI = L // TI
    G_np, pmoh_np = _static_graph(L)
    g = jnp.asarray(G_np)
    pmoh = jnp.asarray(pmoh_np)
    pek = jnp.zeros((_PPAD, P), x.dtype).at[: _MAX_LEN + 1].set(pe_k)
    pev = jnp.zeros((_PPAD, P), x.dtype).at[: _MAX_LEN + 1].set(pe_v)
    mask3 = mask.reshape(B, L * L, 1)
    w1a = W1[:D]
    w1b = W1[D:D + P]
    w1c = W1[D + P:2 * D + P]
    w1d = W1[2 * D + P:]

    full = lambda *shape: pl.BlockSpec(shape, lambda b, t: (0,) * len(shape))
    out = pl.pallas_call(
        functools.partial(_body, TI, L, D, H),
        grid=(B, NI),
        in_specs=[
            pl.BlockSpec((1, L, D), lambda b, t: (b, 0, 0)),
            pl.BlockSpec((1, TI * L, 1), lambda b, t: (b, t, 0)),
            full(_REL_NUM, 2),
            full(_REL_NUM, L, L),
            full(D, D),
            full(D, D),
            full(D, D),
            full(1, D),
            full(_PPAD, P),
            full(_PPAD, P),
            full(D, H),
            full(P, H),
            full(D, H),
            full(P, H),
            full(H, H),
            full(H, 1),
            pl.BlockSpec((TI * L, _PPAD), lambda b, t: (t, 0)),
        ],
        out_specs=pl.BlockSpec((1, TI * L, 1), lambda b, t: (b, t, 0)),
        out_shape=jax.ShapeDtypeStruct((B, L * L, 1), x.dtype),
        scratch_shapes=[
            pltpu.VMEM((L, H), jnp.float32),
            pltpu.VMEM((L, H), jnp.float32),
            pltpu.VMEM((_PPAD, H), jnp.float32),
        ],
    )(x, mask3, comp, g, bases[0], bases[1], root, bias.reshape(1, D),
      pek, pev, w1a, w1b, w1c, w1d, W2, Wp, pmoh)
    return out.reshape(B, L, L)


# speedup vs baseline: 17.6404x; 2.7248x over previous
import functools

import jax
import jax.numpy as jnp
import numpy as np
from jax.experimental import pallas as pl
from jax.experimental.pallas import tpu as pltpu

_WINDOW = 7
_REL_NUM = _WINDOW + 2
_MAX_LEN = 10
_PPAD = 16
_TI = 32

_hdot = functools.partial(jnp.dot, precision=jax.lax.Precision.HIGHEST)


def _bdot(a, b):
    return jnp.dot(a.astype(jnp.bfloat16), b.astype(jnp.bfloat16),
                   preferred_element_type=jnp.float32)


@functools.lru_cache(maxsize=None)
def _static_graph(L, TI):
    i = np.arange(L)[:, None]
    j = np.arange(L)[None, :]
    rel_adj = np.where(j > i, 1, 0).astype(np.int64)
    d = i - j
    lower = -np.minimum(np.ceil(d / 2.0), float(_WINDOW + 1)).astype(np.int64)
    rel_adj = np.where(j < i, lower, rel_adj)
    et = (rel_adj % _REL_NUM).astype(np.int64)
    et_in = et.T
    G = np.zeros((_REL_NUM, L, L), np.float32)
    for r in range(_REL_NUM):
        sel = (et_in == r)
        cnt = np.maximum(sel.sum(axis=1, keepdims=True), 1)
        G[r] = sel / cnt
    pm = np.clip(i - j + 1, 0, _MAX_LEN).reshape(-1)
    rows = np.arange(L * L)
    KE = _PPAD + L + TI
    E = np.zeros((L * L, KE), np.float32)
    E[rows, pm] = 1.0
    E[rows, _PPAD + rows % L] = 1.0
    E[rows, _PPAD + L + (rows // L) % TI] = 1.0
    return G, E


def _body(TI, L, D, H, KE,
          x_ref, mask_ref, comp_ref, g_ref, b0_ref, b1_ref, root_ref,
          bias_ref, pek_ref, pev_ref, w1a_ref, w1b_ref, w1c_ref, w1d_ref,
          w2_ref, wp_ref, e_ref, o_ref, v_s, t_s):
    b = pl.program_id(0)
    t = pl.program_id(1)

    @pl.when((b == 0) & (t == 0))
    def _once():
        v_s[0:_PPAD, :] = (_hdot(pek_ref[:], w1b_ref[:])
                           + _hdot(pev_ref[:], w1d_ref[:]))

    @pl.when(t == 0)
    def _per_batch():
        xb = x_ref[0]
        xb0 = _hdot(xb, b0_ref[:])
        xb1 = _hdot(xb, b1_ref[:])
        m0 = comp_ref[0:1, 0:1] * g_ref[0]
        m1 = comp_ref[0:1, 1:2] * g_ref[0]
        for r in range(1, _REL_NUM):
            m0 = m0 + comp_ref[r:r + 1, 0:1] * g_ref[r]
            m1 = m1 + comp_ref[r:r + 1, 1:2] * g_ref[r]
        outb = (_hdot(m0, xb0) + _hdot(m1, xb1)
                + _hdot(xb, root_ref[:]) + bias_ref[:])
        v_s[_PPAD:_PPAD + L, :] = _hdot(outb, w1a_ref[:])
        t_s[:] = _hdot(outb, w1c_ref[:])

    v_s[_PPAD + L:, :] = t_s[pl.ds(t * TI, TI), :]
    pre = _bdot(e_ref[:], v_s[:])
    h1 = jnp.maximum(pre, 0.0)
    h2 = jnp.maximum(_bdot(h1, w2_ref[:]), 0.0)
    s = _bdot(h2, wp_ref[:])
    o_ref[0] = jax.nn.sigmoid(s) * mask_ref[0]


def kernel(x, mask, pe_k, pe_v, bases, comp, root, bias, W1, W2, Wp):
    B, L, D = x.shape
    H = W2.shape[0]
    P = pe_k.shape[1]
    TI = _TI
    NI = L // TI
    KE = _PPAD + L + TI
    G_np, E_np = _static_graph(L, TI)
    g = jnp.asarray(G_np)
    e = jnp.asarray(E_np).astype(jnp.bfloat16)
    pek = jnp.zeros((_PPAD, P), x.dtype).at[: _MAX_LEN + 1].set(pe_k)
    pev = jnp.zeros((_PPAD, P), x.dtype).at[: _MAX_LEN + 1].set(pe_v)
    mask3 = mask.reshape(B, L * L, 1)
    w1a = W1[:D]
    w1b = W1[D:D + P]
    w1c = W1[D + P:2 * D + P]
    w1d = W1[2 * D + P:]

    full = lambda *shape: pl.BlockSpec(shape, lambda b, t: (0,) * len(shape))
    out = pl.pallas_call(
        functools.partial(_body, TI, L, D, H, KE),
        grid=(B, NI),
        in_specs=[
            pl.BlockSpec((1, L, D), lambda b, t: (b, 0, 0)),
            pl.BlockSpec((1, TI * L, 1), lambda b, t: (b, t, 0)),
            full(_REL_NUM, 2),
            full(_REL_NUM, L, L),
            full(D, D),
            full(D, D),
            full(D, D),
            full(1, D),
            full(_PPAD, P),
            full(_PPAD, P),
            full(D, H),
            full(P, H),
            full(D, H),
            full(P, H),
            full(H, H),
            full(H, 1),
            pl.BlockSpec((TI * L, KE), lambda b, t: (t, 0)),
        ],
        out_specs=pl.BlockSpec((1, TI * L, 1), lambda b, t: (b, t, 0)),
        out_shape=jax.ShapeDtypeStruct((B, L * L, 1), x.dtype),
        scratch_shapes=[
            pltpu.VMEM((KE, H), jnp.float32),
            pltpu.VMEM((L, H), jnp.float32),
        ],
    )(x, mask3, comp, g, bases[0], bases[1], root, bias.reshape(1, D),
      pek, pev, w1a, w1b, w1c, w1d, W2, Wp, e)
    return out.reshape(B, L, L)


# speedup vs baseline: 36.5056x; 2.0694x over previous
import functools

import jax
import jax.numpy as jnp
import numpy as np
from jax.experimental import pallas as pl
from jax.experimental.pallas import tpu as pltpu

_WINDOW = 7
_REL_NUM = _WINDOW + 2
_MAX_LEN = 10
_PPAD = 16
_TI = 32

_hdot = functools.partial(jnp.dot, precision=jax.lax.Precision.HIGHEST)
_bf = jnp.bfloat16


@functools.lru_cache(maxsize=None)
def _static_graph(L, TI):
    i = np.arange(L)[:, None]
    j = np.arange(L)[None, :]
    rel_adj = np.where(j > i, 1, 0).astype(np.int64)
    d = i - j
    lower = -np.minimum(np.ceil(d / 2.0), float(_WINDOW + 1)).astype(np.int64)
    rel_adj = np.where(j < i, lower, rel_adj)
    et = (rel_adj % _REL_NUM).astype(np.int64)
    et_in = et.T
    GT = np.zeros((_REL_NUM, L, L), np.float32)
    for r in range(_REL_NUM):
        sel = (et_in == r)
        cnt = np.maximum(sel.sum(axis=1, keepdims=True), 1)
        GT[r] = (sel / cnt).T
    pm = np.clip(i - j + 1, 0, _MAX_LEN).reshape(-1)
    rows = np.arange(L * L)
    KE = _PPAD + L + TI
    E = np.zeros((L * L, KE), np.float32)
    E[rows, pm] = 1.0
    E[rows, _PPAD + rows % L] = 1.0
    E[rows, _PPAD + L + (rows // L) % TI] = 1.0
    return GT, E.T.copy()


def _body(TI, L, D, H, KE,
          xt_ref, mask_ref, comp_ref, gt_ref, b0t_ref, b1t_ref, roott_ref,
          biasc_ref, pekt_ref, pevt_ref, w1at_ref, w1bt_ref, w1ct_ref,
          w1dt_ref, w2t_ref, wpt_ref, et_ref, o_ref, v_s, t_s):
    b = pl.program_id(0)
    t = pl.program_id(1)

    @pl.when((b == 0) & (t == 0))
    def _once():
        v_s[:, 0:_PPAD] = (_hdot(w1bt_ref[:], pekt_ref[:])
                           + _hdot(w1dt_ref[:], pevt_ref[:])).astype(_bf)

    @pl.when(t == 0)
    def _per_batch():
        xbt = xt_ref[0]
        xb0t = _hdot(b0t_ref[:], xbt)
        xb1t = _hdot(b1t_ref[:], xbt)
        m0t = comp_ref[0:1, 0:1] * gt_ref[0]
        m1t = comp_ref[0:1, 1:2] * gt_ref[0]
        for r in range(1, _REL_NUM):
            m0t = m0t + comp_ref[r:r + 1, 0:1] * gt_ref[r]
            m1t = m1t + comp_ref[r:r + 1, 1:2] * gt_ref[r]
        outt = (_hdot(xb0t, m0t) + _hdot(xb1t, m1t)
                + _hdot(roott_ref[:], xbt) + biasc_ref[:])
        v_s[:, _PPAD:_PPAD + L] = _hdot(w1at_ref[:], outt).astype(_bf)
        tt = _hdot(w1ct_ref[:], outt).astype(_bf)
        for n in range(L // TI):
            t_s[n] = tt[:, n * TI:(n + 1) * TI]

    v_s[:, _PPAD + L:] = t_s[t]
    pre = jnp.dot(v_s[:], et_ref[:], preferred_element_type=jnp.float32)
    h1 = jnp.maximum(pre.astype(_bf), jnp.asarray(0, _bf))
    h2 = jnp.maximum(
        jnp.dot(w2t_ref[:], h1, preferred_element_type=jnp.float32).astype(_bf),
        jnp.asarray(0, _bf))
    s = jnp.dot(wpt_ref[:], h2, preferred_element_type=jnp.float32)
    o_ref[0] = jax.nn.sigmoid(s) * mask_ref[0]


def kernel(x, mask, pe_k, pe_v, bases, comp, root, bias, W1, W2, Wp):
    B, L, D = x.shape
    H = W2.shape[0]
    P = pe_k.shape[1]
    TI = _TI
    NI = L // TI
    KE = _PPAD + L + TI
    GT_np, ET_np = _static_graph(L, TI)
    gt = jnp.asarray(GT_np)
    et = jnp.asarray(ET_np).astype(_bf)
    pekt = jnp.zeros((_PPAD, P), x.dtype).at[: _MAX_LEN + 1].set(pe_k).T
    pevt = jnp.zeros((_PPAD, P), x.dtype).at[: _MAX_LEN + 1].set(pe_v).T
    xt = x.transpose(0, 2, 1)
    mask3 = mask.reshape(B * NI, 1, TI * L)

    full = lambda *shape: pl.BlockSpec(shape, lambda b, t: (0,) * len(shape))
    out = pl.pallas_call(
        functools.partial(_body, TI, L, D, H, KE),
        grid=(B, NI),
        in_specs=[
            pl.BlockSpec((1, D, L), lambda b, t: (b, 0, 0)),
            pl.BlockSpec((1, 1, TI * L), lambda b, t: (b * NI + t, 0, 0)),
            full(_REL_NUM, 2),
            full(_REL_NUM, L, L),
            full(D, D),
            full(D, D),
            full(D, D),
            full(D, 1),
            full(P, _PPAD),
            full(P, _PPAD),
            full(H, D),
            full(H, P),
            full(H, D),
            full(H, P),
            full(H, H),
            full(1, H),
            pl.BlockSpec((KE, TI * L), lambda b, t: (0, t)),
        ],
        out_specs=pl.BlockSpec((1, 1, TI * L), lambda b, t: (b * NI + t, 0, 0)),
        out_shape=jax.ShapeDtypeStruct((B * NI, 1, TI * L), x.dtype),
        scratch_shapes=[
            pltpu.VMEM((H, KE), _bf),
            pltpu.VMEM((NI, H, TI), _bf),
        ],
    )(xt, mask3, comp, gt, bases[0].T, bases[1].T, root.T, bias.reshape(D, 1),
      pekt, pevt, W1[:D].T, W1[D:D + P].T, W1[D + P:2 * D + P].T,
      W1[2 * D + P:].T, W2.T.astype(_bf), Wp.T.astype(_bf), et)
    return out.reshape(B, L, L)


# speedup vs baseline: 42.7158x; 1.1701x over previous
import functools

import jax
import jax.numpy as jnp
import numpy as np
from jax.experimental import pallas as pl
from jax.experimental.pallas import tpu as pltpu

_WINDOW = 7
_REL_NUM = _WINDOW + 2
_MAX_LEN = 10
_PPAD = 16
_TI = 64

_hdot = functools.partial(jnp.dot, precision=jax.lax.Precision.HIGHEST)
_bf = jnp.bfloat16


def _bdot(a, b):
    return jnp.dot(a.astype(_bf), b.astype(_bf),
                   preferred_element_type=jnp.float32)


@functools.lru_cache(maxsize=None)
def _static_graph(L, TI):
    i = np.arange(L)[:, None]
    j = np.arange(L)[None, :]
    rel_adj = np.where(j > i, 1, 0).astype(np.int64)
    d = i - j
    lower = -np.minimum(np.ceil(d / 2.0), float(_WINDOW + 1)).astype(np.int64)
    rel_adj = np.where(j < i, lower, rel_adj)
    et = (rel_adj % _REL_NUM).astype(np.int64)
    et_in = et.T
    GT = np.zeros((_REL_NUM, L, L), np.float32)
    for r in range(_REL_NUM):
        sel = (et_in == r)
        cnt = np.maximum(sel.sum(axis=1, keepdims=True), 1)
        GT[r] = (sel / cnt).T
    pm = np.clip(i - j + 1, 0, _MAX_LEN).reshape(-1)
    rows = np.arange(L * L)
    KE = _PPAD + L + TI
    E = np.zeros((L * L, KE), np.float32)
    E[rows, pm] = 1.0
    E[rows, _PPAD + rows % L] = 1.0
    E[rows, _PPAD + L + (rows // L) % TI] = 1.0
    return GT, E.T.copy()


def _body(TI, L, D, H, KE,
          xt_ref, mask_ref, comp_ref, gt_ref, b0t_ref, b1t_ref, roott_ref,
          biasc_ref, pekt_ref, pevt_ref, w1at_ref, w1bt_ref, w1ct_ref,
          w1dt_ref, w2t_ref, wpt_ref, et_ref, o_ref, v_s, t_s):
    b = pl.program_id(0)
    t = pl.program_id(1)

    @pl.when((b == 0) & (t == 0))
    def _once():
        v_s[:, 0:_PPAD] = (_hdot(w1bt_ref[:], pekt_ref[:])
                           + _hdot(w1dt_ref[:], pevt_ref[:])).astype(_bf)

    @pl.when(t == 0)
    def _per_batch():
        xbt = xt_ref[0]
        xb0t = _bdot(b0t_ref[:], xbt)
        xb1t = _bdot(b1t_ref[:], xbt)
        m0t = comp_ref[0:1, 0:1] * gt_ref[0]
        m1t = comp_ref[0:1, 1:2] * gt_ref[0]
        for r in range(1, _REL_NUM):
            m0t = m0t + comp_ref[r:r + 1, 0:1] * gt_ref[r]
            m1t = m1t + comp_ref[r:r + 1, 1:2] * gt_ref[r]
        outt = (_bdot(xb0t, m0t) + _bdot(xb1t, m1t)
                + _bdot(roott_ref[:], xbt) + biasc_ref[:])
        v_s[:, _PPAD:_PPAD + L] = _bdot(w1at_ref[:], outt).astype(_bf)
        tt = _bdot(w1ct_ref[:], outt).astype(_bf)
        for n in range(L // TI):
            t_s[n] = tt[:, n * TI:(n + 1) * TI]

    v_s[:, _PPAD + L:] = t_s[t]
    pre = jnp.dot(v_s[:], et_ref[:], preferred_element_type=jnp.float32)
    h1 = jnp.maximum(pre.astype(_bf), jnp.asarray(0, _bf))
    h2 = jnp.maximum(
        jnp.dot(w2t_ref[:], h1, preferred_element_type=jnp.float32).astype(_bf),
        jnp.asarray(0, _bf))
    s = jnp.dot(wpt_ref[:], h2, preferred_element_type=jnp.float32)
    o_ref[0] = jax.nn.sigmoid(s) * mask_ref[0]


def kernel(x, mask, pe_k, pe_v, bases, comp, root, bias, W1, W2, Wp):
    B, L, D = x.shape
    H = W2.shape[0]
    P = pe_k.shape[1]
    TI = _TI
    NI = L // TI
    KE = _PPAD + L + TI
    GT_np, ET_np = _static_graph(L, TI)
    gt = jnp.asarray(GT_np)
    et = jnp.asarray(ET_np).astype(_bf)
    pekt = jnp.zeros((_PPAD, P), x.dtype).at[: _MAX_LEN + 1].set(pe_k).T
    pevt = jnp.zeros((_PPAD, P), x.dtype).at[: _MAX_LEN + 1].set(pe_v).T
    xt = x.transpose(0, 2, 1)
    mask3 = mask.reshape(B * NI, 1, TI * L)

    full = lambda *shape: pl.BlockSpec(shape, lambda b, t: (0,) * len(shape))
    out = pl.pallas_call(
        functools.partial(_body, TI, L, D, H, KE),
        grid=(B, NI),
        in_specs=[
            pl.BlockSpec((1, D, L), lambda b, t: (b, 0, 0)),
            pl.BlockSpec((1, 1, TI * L), lambda b, t: (b * NI + t, 0, 0)),
            full(_REL_NUM, 2),
            full(_REL_NUM, L, L),
            full(D, D),
            full(D, D),
            full(D, D),
            full(D, 1),
            full(P, _PPAD),
            full(P, _PPAD),
            full(H, D),
            full(H, P),
            full(H, D),
            full(H, P),
            full(H, H),
            full(1, H),
            pl.BlockSpec((KE, TI * L), lambda b, t: (0, t)),
        ],
        out_specs=pl.BlockSpec((1, 1, TI * L), lambda b, t: (b * NI + t, 0, 0)),
        out_shape=jax.ShapeDtypeStruct((B * NI, 1, TI * L), x.dtype),
        scratch_shapes=[
            pltpu.VMEM((H, KE), _bf),
            pltpu.VMEM((NI, H, TI), _bf),
        ],
    )(xt, mask3, comp, gt, bases[0].T, bases[1].T, root.T, bias.reshape(D, 1),
      pekt, pevt, W1[:D].T, W1[D:D + P].T, W1[D + P:2 * D + P].T,
      W1[2 * D + P:].T, W2.T.astype(_bf), Wp.T.astype(_bf), et)
    return out.reshape(B, L, L)


# speedup vs baseline: 42.9487x; 1.0055x over previous
import functools

import jax
import jax.numpy as jnp
import numpy as np
from jax.experimental import pallas as pl
from jax.experimental.pallas import tpu as pltpu

_WINDOW = 7
_REL_NUM = _WINDOW + 2
_MAX_LEN = 10
_PPAD = 16
_TI = 64

_hdot = functools.partial(jnp.dot, precision=jax.lax.Precision.HIGHEST)
_bf = jnp.bfloat16


def _bdot(a, b):
    return jnp.dot(a.astype(_bf), b.astype(_bf),
                   preferred_element_type=jnp.float32)


@functools.lru_cache(maxsize=None)
def _static_graph(L, TI):
    i = np.arange(L)[:, None]
    j = np.arange(L)[None, :]
    rel_adj = np.where(j > i, 1, 0).astype(np.int64)
    d = i - j
    lower = -np.minimum(np.ceil(d / 2.0), float(_WINDOW + 1)).astype(np.int64)
    rel_adj = np.where(j < i, lower, rel_adj)
    et = (rel_adj % _REL_NUM).astype(np.int64)
    et_in = et.T
    GT = np.zeros((_REL_NUM, L, L), np.float32)
    for r in range(_REL_NUM):
        sel = (et_in == r)
        cnt = np.maximum(sel.sum(axis=1, keepdims=True), 1)
        GT[r] = (sel / cnt).T
    pm = np.clip(i - j + 1, 0, _MAX_LEN).reshape(-1)
    rows = np.arange(L * L)
    KE = _PPAD + L + TI
    E = np.zeros((L * L, KE), np.float32)
    E[rows, pm] = 1.0
    E[rows, _PPAD + rows % L] = 1.0
    E[rows, _PPAD + L + (rows // L) % TI] = 1.0
    return GT, E.T.copy()


def _body(TI, L, D, H, KE,
          xt_ref, mask_ref, comp_ref, gt_ref, b0t_ref, b1t_ref, roott_ref,
          biasc_ref, pekt_ref, pevt_ref, w1at_ref, w1bt_ref, w1ct_ref,
          w1dt_ref, w2t_ref, wpt_ref, et_ref, o_ref, v_s, t_s):
    t = pl.program_id(1)

    @pl.when(t == 0)
    def _per_batch():
        v_s[:, 0:_PPAD] = (_bdot(w1bt_ref[:], pekt_ref[:])
                           + _bdot(w1dt_ref[:], pevt_ref[:])).astype(_bf)
        xbt = xt_ref[0]
        xb0t = _bdot(b0t_ref[:], xbt)
        xb1t = _bdot(b1t_ref[:], xbt)
        m0t = comp_ref[0:1, 0:1] * gt_ref[0]
        m1t = comp_ref[0:1, 1:2] * gt_ref[0]
        for r in range(1, _REL_NUM):
            m0t = m0t + comp_ref[r:r + 1, 0:1] * gt_ref[r]
            m1t = m1t + comp_ref[r:r + 1, 1:2] * gt_ref[r]
        outt = (_bdot(xb0t, m0t) + _bdot(xb1t, m1t)
                + _bdot(roott_ref[:], xbt) + biasc_ref[:])
        v_s[:, _PPAD:_PPAD + L] = _bdot(w1at_ref[:], outt).astype(_bf)
        tt = _bdot(w1ct_ref[:], outt).astype(_bf)
        for n in range(L // TI):
            t_s[n] = tt[:, n * TI:(n + 1) * TI]

    v_s[:, _PPAD + L:] = t_s[t]
    pre = jnp.dot(v_s[:], et_ref[:], preferred_element_type=jnp.float32)
    h1 = jnp.maximum(pre.astype(_bf), jnp.asarray(0, _bf))
    h2 = jnp.maximum(
        jnp.dot(w2t_ref[:], h1, preferred_element_type=jnp.float32).astype(_bf),
        jnp.asarray(0, _bf))
    s = jnp.dot(wpt_ref[:], h2, preferred_element_type=jnp.float32)
    o_ref[0] = jax.nn.sigmoid(s) * mask_ref[0]


def kernel(x, mask, pe_k, pe_v, bases, comp, root, bias, W1, W2, Wp):
    B, L, D = x.shape
    H = W2.shape[0]
    P = pe_k.shape[1]
    TI = _TI
    NI = L // TI
    KE = _PPAD + L + TI
    GT_np, ET_np = _static_graph(L, TI)
    gt = jnp.asarray(GT_np)
    et = jnp.asarray(ET_np).astype(_bf)
    pekt = jnp.zeros((_PPAD, P), x.dtype).at[: _MAX_LEN + 1].set(pe_k).T
    pevt = jnp.zeros((_PPAD, P), x.dtype).at[: _MAX_LEN + 1].set(pe_v).T
    xt = x.transpose(0, 2, 1)
    mask3 = mask.reshape(B * NI, 1, TI * L)

    full = lambda *shape: pl.BlockSpec(shape, lambda b, t: (0,) * len(shape))
    out = pl.pallas_call(
        functools.partial(_body, TI, L, D, H, KE),
        grid=(B, NI),
        in_specs=[
            pl.BlockSpec((1, D, L), lambda b, t: (b, 0, 0)),
            pl.BlockSpec((1, 1, TI * L), lambda b, t: (b * NI + t, 0, 0)),
            full(_REL_NUM, 2),
            full(_REL_NUM, L, L),
            full(D, D),
            full(D, D),
            full(D, D),
            full(D, 1),
            full(P, _PPAD),
            full(P, _PPAD),
            full(H, D),
            full(H, P),
            full(H, D),
            full(H, P),
            full(H, H),
            full(1, H),
            pl.BlockSpec((KE, TI * L), lambda b, t: (0, t)),
        ],
        out_specs=pl.BlockSpec((1, 1, TI * L), lambda b, t: (b * NI + t, 0, 0)),
        out_shape=jax.ShapeDtypeStruct((B * NI, 1, TI * L), x.dtype),
        compiler_params=pltpu.CompilerParams(
            dimension_semantics=("parallel", "arbitrary")),
        scratch_shapes=[
            pltpu.VMEM((H, KE), _bf),
            pltpu.VMEM((NI, H, TI), _bf),
        ],
    )(xt, mask3, comp, gt, bases[0].T, bases[1].T, root.T, bias.reshape(D, 1),
      pekt, pevt, W1[:D].T, W1[D:D + P].T, W1[D + P:2 * D + P].T,
      W1[2 * D + P:].T, W2.T.astype(_bf), Wp.T.astype(_bf), et)
    return out.reshape(B, L, L)


# speedup vs baseline: 48.1680x; 1.1215x over previous
import functools

import jax
import jax.numpy as jnp
import numpy as np
from jax.experimental import pallas as pl
from jax.experimental.pallas import tpu as pltpu

_WINDOW = 7
_REL_NUM = _WINDOW + 2
_MAX_LEN = 10
_PPAD = 16
_TI = 64

_bf = jnp.bfloat16


def _bdot(a, b):
    return jnp.dot(a.astype(_bf), b.astype(_bf),
                   preferred_element_type=jnp.float32)


def _bdot_tl(w, y):
    return jax.lax.dot_general(
        w.astype(_bf), y.astype(_bf), (((0,), (0,)), ((), ())),
        preferred_element_type=jnp.float32)


@functools.lru_cache(maxsize=None)
def _static_graph(L, TI):
    i = np.arange(L)[:, None]
    j = np.arange(L)[None, :]
    rel_adj = np.where(j > i, 1, 0).astype(np.int64)
    d = i - j
    lower = -np.minimum(np.ceil(d / 2.0), float(_WINDOW + 1)).astype(np.int64)
    rel_adj = np.where(j < i, lower, rel_adj)
    et = (rel_adj % _REL_NUM).astype(np.int64)
    et_in = et.T
    GT = np.zeros((_REL_NUM, L, L), np.float32)
    for r in range(_REL_NUM):
        sel = (et_in == r)
        cnt = np.maximum(sel.sum(axis=1, keepdims=True), 1)
        GT[r] = (sel / cnt).T
    pm = np.clip(i - j + 1, 0, _MAX_LEN).reshape(-1)
    rows = np.arange(L * L)
    KE = _PPAD + L + TI
    E = np.zeros((L * L, KE), np.float32)
    E[rows, pm] = 1.0
    E[rows, _PPAD + rows % L] = 1.0
    E[rows, _PPAD + L + (rows // L) % TI] = 1.0
    return GT, E.T.copy()


def _body(TI, L, D, H, KE, P,
          x_ref, mask_ref, comp_ref, gt_ref, bases_ref, root_ref,
          biasc_ref, pekp_ref, pevp_ref, w1_ref, w2_ref, wp_ref,
          et_ref, o_ref, v_s, t_s):
    t = pl.program_id(1)

    @pl.when(t == 0)
    def _per_batch():
        v_s[:, 0:_PPAD] = (_bdot_tl(w1_ref[D:D + P], pekp_ref[:])
                           + _bdot_tl(w1_ref[2 * D + P:], pevp_ref[:])
                           ).astype(_bf)
        xbt = x_ref[0].T
        xb0t = _bdot_tl(bases_ref[0], xbt)
        xb1t = _bdot_tl(bases_ref[1], xbt)
        m0t = comp_ref[0:1, 0:1] * gt_ref[0]
        m1t = comp_ref[0:1, 1:2] * gt_ref[0]
        for r in range(1, _REL_NUM):
            m0t = m0t + comp_ref[r:r + 1, 0:1] * gt_ref[r]
            m1t = m1t + comp_ref[r:r + 1, 1:2] * gt_ref[r]
        outt = (_bdot(xb0t, m0t) + _bdot(xb1t, m1t)
                + _bdot_tl(root_ref[:], xbt) + biasc_ref[:])
        v_s[:, _PPAD:_PPAD + L] = _bdot_tl(w1_ref[0:D], outt).astype(_bf)
        tt = _bdot_tl(w1_ref[D + P:2 * D + P], outt).astype(_bf)
        for n in range(L // TI):
            t_s[n] = tt[:, n * TI:(n + 1) * TI]

    v_s[:, _PPAD + L:] = t_s[t]
    pre = jnp.dot(v_s[:], et_ref[:], preferred_element_type=jnp.float32)
    h1 = jnp.maximum(pre.astype(_bf), jnp.asarray(0, _bf))
    h2 = jnp.maximum(_bdot_tl(w2_ref[:], h1).astype(_bf), jnp.asarray(0, _bf))
    s = _bdot_tl(wp_ref[:], h2)
    o_ref[0] = jax.nn.sigmoid(s) * mask_ref[0]


def kernel(x, mask, pe_k, pe_v, bases, comp, root, bias, W1, W2, Wp):
    B, L, D = x.shape
    H = W2.shape[0]
    P = pe_k.shape[1]
    TI = _TI
    NI = L // TI
    KE = _PPAD + L + TI
    GT_np, ET_np = _static_graph(L, TI)
    gt = jnp.asarray(GT_np)
    et = jnp.asarray(ET_np).astype(_bf)
    pekp = jnp.zeros((_PPAD, P), x.dtype).at[: _MAX_LEN + 1].set(pe_k).T
    pevp = jnp.zeros((_PPAD, P), x.dtype).at[: _MAX_LEN + 1].set(pe_v).T
    mask3 = mask.reshape(B * NI, 1, TI * L)

    full = lambda *shape: pl.BlockSpec(shape, lambda b, t: (0,) * len(shape))
    out = pl.pallas_call(
        functools.partial(_body, TI, L, D, H, KE, P),
        grid=(B, NI),
        in_specs=[
            pl.BlockSpec((1, L, D), lambda b, t: (b, 0, 0)),
            pl.BlockSpec((1, 1, TI * L), lambda b, t: (b * NI + t, 0, 0)),
            full(_REL_NUM, 2),
            full(_REL_NUM, L, L),
            full(2, D, D),
            full(D, D),
            full(D, 1),
            full(P, _PPAD),
            full(P, _PPAD),
            full(2 * D + 2 * P, H),
            full(H, H),
            full(H, 1),
            pl.BlockSpec((KE, TI * L), lambda b, t: (0, t)),
        ],
        out_specs=pl.BlockSpec((1, 1, TI * L), lambda b, t: (b * NI + t, 0, 0)),
        out_shape=jax.ShapeDtypeStruct((B * NI, 1, TI * L), x.dtype),
        compiler_params=pltpu.CompilerParams(
            dimension_semantics=("parallel", "arbitrary")),
        scratch_shapes=[
            pltpu.VMEM((H, KE), _bf),
            pltpu.VMEM((NI, H, TI), _bf),
        ],
    )(x, mask3, comp, gt, bases, root, bias.reshape(D, 1),
      pekp, pevp, W1, W2, Wp, et)
    return out.reshape(B, L, L)


# speedup vs baseline: 50.7749x; 1.0541x over previous
import functools

import jax
import jax.numpy as jnp
import numpy as np
from jax.experimental import pallas as pl
from jax.experimental.pallas import tpu as pltpu

_WINDOW = 7
_REL_NUM = _WINDOW + 2
_MAX_LEN = 10
_PPAD = 16
_TI = 64

_bf = jnp.bfloat16


def _bdot(a, b):
    return jnp.dot(a.astype(_bf), b.astype(_bf),
                   preferred_element_type=jnp.float32)


def _bdot_tl(w, y):
    return jax.lax.dot_general(
        w.astype(_bf), y.astype(_bf), (((0,), (0,)), ((), ())),
        preferred_element_type=jnp.float32)


def _bdot_tt(w, y):
    return jax.lax.dot_general(
        w.astype(_bf), y.astype(_bf), (((0,), (1,)), ((), ())),
        preferred_element_type=jnp.float32)


@functools.lru_cache(maxsize=None)
def _static_graph(L, TI):
    i = np.arange(L)[:, None]
    j = np.arange(L)[None, :]
    rel_adj = np.where(j > i, 1, 0).astype(np.int64)
    d = i - j
    lower = -np.minimum(np.ceil(d / 2.0), float(_WINDOW + 1)).astype(np.int64)
    rel_adj = np.where(j < i, lower, rel_adj)
    et = (rel_adj % _REL_NUM).astype(np.int64)
    et_in = et.T
    GT = np.zeros((_REL_NUM, L, L), np.float32)
    for r in range(_REL_NUM):
        sel = (et_in == r)
        cnt = np.maximum(sel.sum(axis=1, keepdims=True), 1)
        GT[r] = (sel / cnt).T
    pm = np.clip(i - j + 1, 0, _MAX_LEN).reshape(-1)
    rows = np.arange(L * L)
    KE = _PPAD + L + TI
    E = np.zeros((L * L, KE), np.float32)
    E[rows, pm] = 1.0
    E[rows, _PPAD + rows % L] = 1.0
    E[rows, _PPAD + L + (rows // L) % TI] = 1.0
    return GT, E.T.copy()


def _body(TI, L, D, H, KE, P,
          x_ref, mask_ref, comp_ref, gt_ref, bases_ref, root_ref,
          biasc_ref, pekp_ref, pevp_ref, w1_ref, w2_ref, wp_ref,
          et_ref, o_ref, v_s, t_s):
    t = pl.program_id(1)

    @pl.when(t == 0)
    def _per_batch():
        v_s[:, 0:_PPAD] = jnp.zeros((H, _PPAD), _bf)
        v_s[:, 0:_MAX_LEN + 1] = (_bdot_tt(w1_ref[D:D + P], pekp_ref[:])
                                  + _bdot_tt(w1_ref[2 * D + P:], pevp_ref[:])
                                  ).astype(_bf)
        xbt = x_ref[0].T
        xb0t = _bdot_tl(bases_ref[0], xbt)
        xb1t = _bdot_tl(bases_ref[1], xbt)
        m0t = comp_ref[0:1, 0:1] * gt_ref[0]
        m1t = comp_ref[0:1, 1:2] * gt_ref[0]
        for r in range(1, _REL_NUM):
            m0t = m0t + comp_ref[r:r + 1, 0:1] * gt_ref[r]
            m1t = m1t + comp_ref[r:r + 1, 1:2] * gt_ref[r]
        outt = (_bdot(xb0t, m0t) + _bdot(xb1t, m1t)
                + _bdot_tl(root_ref[:], xbt) + biasc_ref[:])
        v_s[:, _PPAD:_PPAD + L] = _bdot_tl(w1_ref[0:D], outt).astype(_bf)
        tt = _bdot_tl(w1_ref[D + P:2 * D + P], outt).astype(_bf)
        for n in range(L // TI):
            t_s[n] = tt[:, n * TI:(n + 1) * TI]

    v_s[:, _PPAD + L:] = t_s[t]
    pre = jnp.dot(v_s[:], et_ref[:], preferred_element_type=jnp.float32)
    h1 = jnp.maximum(pre.astype(_bf), jnp.asarray(0, _bf))
    h2 = jnp.maximum(_bdot_tl(w2_ref[:], h1).astype(_bf), jnp.asarray(0, _bf))
    s = _bdot_tl(wp_ref[:], h2)
    o_ref[0] = jax.nn.sigmoid(s) * mask_ref[0]


def kernel(x, mask, pe_k, pe_v, bases, comp, root, bias, W1, W2, Wp):
    B, L, D = x.shape
    H = W2.shape[0]
    P = pe_k.shape[1]
    TI = _TI
    NI = L // TI
    KE = _PPAD + L + TI
    GT_np, ET_np = _static_graph(L, TI)
    gt = jnp.asarray(GT_np)
    et = jnp.asarray(ET_np).astype(_bf)
    mask3 = mask.reshape(B * NI, 1, TI * L)

    full = lambda *shape: pl.BlockSpec(shape, lambda b, t: (0,) * len(shape))
    out = pl.pallas_call(
        functools.partial(_body, TI, L, D, H, KE, P),
        grid=(B, NI),
        in_specs=[
            pl.BlockSpec((1, L, D), lambda b, t: (b, 0, 0)),
            pl.BlockSpec((1, 1, TI * L), lambda b, t: (b * NI + t, 0, 0)),
            full(_REL_NUM, 2),
            full(_REL_NUM, L, L),
            full(2, D, D),
            full(D, D),
            full(D, 1),
            full(_MAX_LEN + 1, P),
            full(_MAX_LEN + 1, P),
            full(2 * D + 2 * P, H),
            full(H, H),
            full(H, 1),
            pl.BlockSpec((KE, TI * L), lambda b, t: (0, t)),
        ],
        out_specs=pl.BlockSpec((1, 1, TI * L), lambda b, t: (b * NI + t, 0, 0)),
        out_shape=jax.ShapeDtypeStruct((B * NI, 1, TI * L), x.dtype),
        compiler_params=pltpu.CompilerParams(
            dimension_semantics=("parallel", "arbitrary")),
        scratch_shapes=[
            pltpu.VMEM((H, KE), _bf),
            pltpu.VMEM((NI, H, TI), _bf),
        ],
    )(x, mask3, comp, gt, bases, root, bias.reshape(D, 1),
      pe_k, pe_v, W1, W2, Wp, et)
    return out.reshape(B, L, L)


# speedup vs baseline: 52.5025x; 1.0340x over previous
import functools

import jax
import jax.numpy as jnp
import numpy as np
from jax.experimental import pallas as pl
from jax.experimental.pallas import tpu as pltpu

_WINDOW = 7
_REL_NUM = _WINDOW + 2
_MAX_LEN = 10
_PPAD = 16
_TI = 64

_bf = jnp.bfloat16


def _bdot(a, b):
    return jnp.dot(a.astype(_bf), b.astype(_bf),
                   preferred_element_type=jnp.float32)


def _bdot_tl(w, y):
    return jax.lax.dot_general(
        w.astype(_bf), y.astype(_bf), (((0,), (0,)), ((), ())),
        preferred_element_type=jnp.float32)


def _bdot_tt(w, y):
    return jax.lax.dot_general(
        w.astype(_bf), y.astype(_bf), (((0,), (1,)), ((), ())),
        preferred_element_type=jnp.float32)


@functools.lru_cache(maxsize=None)
def _static_graph(L, TI):
    i = np.arange(L)[:, None]
    j = np.arange(L)[None, :]
    rel_adj = np.where(j > i, 1, 0).astype(np.int64)
    d = i - j
    lower = -np.minimum(np.ceil(d / 2.0), float(_WINDOW + 1)).astype(np.int64)
    rel_adj = np.where(j < i, lower, rel_adj)
    et = (rel_adj % _REL_NUM).astype(np.int64)
    et_in = et.T
    GT = np.zeros((_REL_NUM, L, L), np.float32)
    for r in range(_REL_NUM):
        sel = (et_in == r)
        cnt = np.maximum(sel.sum(axis=1, keepdims=True), 1)
        GT[r] = (sel / cnt).T
    pm = np.clip(i - j + 1, 0, _MAX_LEN).reshape(-1)
    rows = np.arange(L * L)
    KE = _PPAD + L + TI
    E = np.zeros((L * L, KE), np.float32)
    E[rows, pm] = 1.0
    E[rows, _PPAD + rows % L] = 1.0
    E[rows, _PPAD + L + (rows // L) % TI] = 1.0
    return GT, E.T.copy()


def _body(TI, L, D, H, KE, P,
          x_ref, mask_ref, comp_ref, gt_ref, bases_ref, root_ref,
          biasc_ref, pekp_ref, pevp_ref, w1_ref, w2_ref, wp_ref,
          et_ref, o_ref, v_s, t_s):
    t = pl.program_id(1)

    @pl.when(t == 0)
    def _per_batch():
        v_s[:, 0:_PPAD] = jnp.zeros((H, _PPAD), _bf)
        v_s[:, 0:_MAX_LEN + 1] = (_bdot_tt(w1_ref[D:D + P], pekp_ref[:])
                                  + _bdot_tt(w1_ref[2 * D + P:], pevp_ref[:])
                                  ).astype(_bf)
        xbt = x_ref[:].T
        xb0t = _bdot_tl(bases_ref[0:D], xbt)
        xb1t = _bdot_tl(bases_ref[D:2 * D], xbt)
        m0t = comp_ref[0:1, 0:1] * gt_ref[0]
        m1t = comp_ref[0:1, 1:2] * gt_ref[0]
        for r in range(1, _REL_NUM):
            m0t = m0t + comp_ref[0:1, 2 * r:2 * r + 1] * gt_ref[r]
            m1t = m1t + comp_ref[0:1, 2 * r + 1:2 * r + 2] * gt_ref[r]
        outt = (_bdot(xb0t, m0t) + _bdot(xb1t, m1t)
                + _bdot_tl(root_ref[:], xbt) + biasc_ref[:])
        v_s[:, _PPAD:_PPAD + L] = _bdot_tl(w1_ref[0:D], outt).astype(_bf)
        tt = _bdot_tl(w1_ref[D + P:2 * D + P], outt).astype(_bf)
        for n in range(L // TI):
            t_s[n] = tt[:, n * TI:(n + 1) * TI]

    v_s[:, _PPAD + L:] = t_s[t]
    pre = jnp.dot(v_s[:], et_ref[:], preferred_element_type=jnp.float32)
    h1 = jnp.maximum(pre.astype(_bf), jnp.asarray(0, _bf))
    h2 = jnp.maximum(_bdot_tl(w2_ref[:], h1).astype(_bf), jnp.asarray(0, _bf))
    s = _bdot(wp_ref[:], h2)
    o_ref[0] = jax.nn.sigmoid(s) * mask_ref[0]


def kernel(x, mask, pe_k, pe_v, bases, comp, root, bias, W1, W2, Wp):
    B, L, D = x.shape
    H = W2.shape[0]
    P = pe_k.shape[1]
    TI = _TI
    NI = L // TI
    KE = _PPAD + L + TI
    GT_np, ET_np = _static_graph(L, TI)
    gt = jnp.asarray(GT_np)
    et = jnp.asarray(ET_np).astype(_bf)
    mask3 = mask.reshape(B * NI, 1, TI * L)

    full = lambda *shape: pl.BlockSpec(shape, lambda b, t: (0,) * len(shape))
    out = pl.pallas_call(
        functools.partial(_body, TI, L, D, H, KE, P),
        grid=(B, NI),
        in_specs=[
            pl.BlockSpec((L, D), lambda b, t: (b, 0)),
            pl.BlockSpec((1, 1, TI * L), lambda b, t: (b * NI + t, 0, 0)),
            full(1, 2 * _REL_NUM),
            full(_REL_NUM, L, L),
            full(2 * D, D),
            full(D, D),
            full(D, 1),
            full(_MAX_LEN + 1, P),
            full(_MAX_LEN + 1, P),
            full(2 * D + 2 * P, H),
            full(H, H),
            full(1, H),
            pl.BlockSpec((KE, TI * L), lambda b, t: (0, t)),
        ],
        out_specs=pl.BlockSpec((1, 1, TI * L), lambda b, t: (b * NI + t, 0, 0)),
        out_shape=jax.ShapeDtypeStruct((B * NI, 1, TI * L), x.dtype),
        compiler_params=pltpu.CompilerParams(
            dimension_semantics=("parallel", "arbitrary")),
        scratch_shapes=[
            pltpu.VMEM((H, KE), _bf),
            pltpu.VMEM((NI, H, TI), _bf),
        ],
    )(x.reshape(B * L, D), mask3, comp.reshape(1, 2 * _REL_NUM), gt,
      bases.reshape(2 * D, D), root, bias.reshape(D, 1),
      pe_k, pe_v, W1, W2, Wp.reshape(1, H), et)
    return out.reshape(B, L, L)


# speedup vs baseline: 56.2479x; 1.0713x over previous
import functools

import jax
import jax.numpy as jnp
import numpy as np
from jax.experimental import pallas as pl
from jax.experimental.pallas import tpu as pltpu

_WINDOW = 7
_REL_NUM = _WINDOW + 2
_MAX_LEN = 10
_PPAD = 16
_TI = 64

_bf = jnp.bfloat16


def _bdot(a, b):
    return jnp.dot(a.astype(_bf), b.astype(_bf),
                   preferred_element_type=jnp.float32)


def _bdot_tl(w, y):
    return jax.lax.dot_general(
        w.astype(_bf), y.astype(_bf), (((0,), (0,)), ((), ())),
        preferred_element_type=jnp.float32)


def _bdot_tt(w, y):
    return jax.lax.dot_general(
        w.astype(_bf), y.astype(_bf), (((0,), (1,)), ((), ())),
        preferred_element_type=jnp.float32)


@functools.lru_cache(maxsize=None)
def _static_graph(L, TI):
    i = np.arange(L)[:, None]
    j = np.arange(L)[None, :]
    rel_adj = np.where(j > i, 1, 0).astype(np.int64)
    d = i - j
    lower = -np.minimum(np.ceil(d / 2.0), float(_WINDOW + 1)).astype(np.int64)
    rel_adj = np.where(j < i, lower, rel_adj)
    et = (rel_adj % _REL_NUM).astype(np.int64)
    et_in = et.T
    GT = np.zeros((_REL_NUM, L, L), np.float32)
    for r in range(_REL_NUM):
        sel = (et_in == r)
        cnt = np.maximum(sel.sum(axis=1, keepdims=True), 1)
        GT[r] = (sel / cnt).T
    pm = np.clip(i - j + 1, 0, _MAX_LEN).reshape(-1)
    rows = np.arange(L * L)
    KE = _PPAD + L + TI
    E = np.zeros((L * L, KE), np.float32)
    E[rows, pm] = 1.0
    E[rows, _PPAD + rows % L] = 1.0
    E[rows, _PPAD + L + (rows // L) % TI] = 1.0
    return GT, E.T.copy()


def _body(TI, L, D, H, KE, P, BL,
          xcat_ref, mask_ref, comp_ref, gt_ref,
          biasc_ref, pekp_ref, pevp_ref, w1_ref, w2_ref, wp_ref,
          et_ref, o_ref, v_s, t_s):
    b = pl.program_id(0)
    t = pl.program_id(1)

    @pl.when(t == 0)
    def _per_batch():
        v_s[:, 0:_PPAD] = jnp.zeros((H, _PPAD), _bf)
        v_s[:, 0:_MAX_LEN + 1] = (_bdot_tt(w1_ref[D:D + P], pekp_ref[:])
                                  + _bdot_tt(w1_ref[2 * D + P:], pevp_ref[:])
                                  ).astype(_bf)
        xbt = xcat_ref[pl.ds(b * L, L), :].T
        xb0t = _bdot_tl(xcat_ref[BL:BL + D], xbt)
        xb1t = _bdot_tl(xcat_ref[BL + D:BL + 2 * D], xbt)
        m0t = comp_ref[0:1, 0:1] * gt_ref[0]
        m1t = comp_ref[0:1, 1:2] * gt_ref[0]
        for r in range(1, _REL_NUM):
            m0t = m0t + comp_ref[0:1, 2 * r:2 * r + 1] * gt_ref[r]
            m1t = m1t + comp_ref[0:1, 2 * r + 1:2 * r + 2] * gt_ref[r]
        outt = (_bdot(xb0t, m0t) + _bdot(xb1t, m1t)
                + _bdot_tl(xcat_ref[BL + 2 * D:BL + 3 * D], xbt)
                + biasc_ref[:])
        v_s[:, _PPAD:_PPAD + L] = _bdot_tl(w1_ref[0:D], outt).astype(_bf)
        tt = _bdot_tl(w1_ref[D + P:2 * D + P], outt).astype(_bf)
        for n in range(L // TI):
            t_s[n] = tt[:, n * TI:(n + 1) * TI]

    v_s[:, _PPAD + L:] = t_s[t]
    pre = jnp.dot(v_s[:], et_ref[:], preferred_element_type=jnp.float32)
    h1 = jnp.maximum(pre.astype(_bf), jnp.asarray(0, _bf))
    h2 = jnp.maximum(_bdot_tl(w2_ref[:], h1).astype(_bf), jnp.asarray(0, _bf))
    s = _bdot(wp_ref[:], h2)
    o_ref[0] = jax.nn.sigmoid(s) * mask_ref[0]


def kernel(x, mask, pe_k, pe_v, bases, comp, root, bias, W1, W2, Wp):
    B, L, D = x.shape
    H = W2.shape[0]
    P = pe_k.shape[1]
    TI = _TI
    NI = L // TI
    KE = _PPAD + L + TI
    GT_np, ET_np = _static_graph(L, TI)
    gt = jnp.asarray(GT_np)
    et = jnp.asarray(ET_np).astype(_bf)
    mask3 = mask.reshape(B * NI, 1, TI * L)

    full = lambda *shape: pl.BlockSpec(shape, lambda b, t: (0,) * len(shape))
    out = pl.pallas_call(
        functools.partial(_body, TI, L, D, H, KE, P, B * L),
        grid=(B, NI),
        in_specs=[
            full(B * L + 3 * D, D),
            pl.BlockSpec((1, 1, TI * L), lambda b, t: (b * NI + t, 0, 0)),
            full(1, 2 * _REL_NUM),
            full(_REL_NUM, L, L),
            full(D, 1),
            full(_MAX_LEN + 1, P),
            full(_MAX_LEN + 1, P),
            full(2 * D + 2 * P, H),
            full(H, H),
            full(1, H),
            pl.BlockSpec((KE, TI * L), lambda b, t: (0, t)),
        ],
        out_specs=pl.BlockSpec((1, 1, TI * L), lambda b, t: (b * NI + t, 0, 0)),
        out_shape=jax.ShapeDtypeStruct((B * NI, 1, TI * L), x.dtype),
        compiler_params=pltpu.CompilerParams(
            dimension_semantics=("parallel", "arbitrary")),
        scratch_shapes=[
            pltpu.VMEM((H, KE), _bf),
            pltpu.VMEM((NI, H, TI), _bf),
        ],
    )(jnp.concatenate([x.reshape(B * L, D), bases.reshape(2 * D, D), root],
                      axis=0),
      mask3, comp.reshape(1, 2 * _REL_NUM), gt, bias.reshape(D, 1),
      pe_k, pe_v, W1, W2, Wp.reshape(1, H), et)
    return out.reshape(B, L, L)
